# Initial kernel scaffold; baseline (speedup 1.0000x reference)
import functools

import jax
import jax.numpy as jnp
from jax import lax
from jax.experimental import pallas as pl
from jax.experimental.pallas import tpu as pltpu
from jax.experimental.pallas import tpu_sc as plsc

N = 10000
E = 320000
R = 8
D = 128
G = 64
NB = 4
NTILES = 32
EPT = E // NTILES
FC = EPT // 128
TV = EPT - FC * 128
NCH = FC + 1
KT = R * N + 128
SL = KT // 16

_MESH = plsc.VectorSubcoreMesh(core_axis_name="c", subcore_axis_name="s")


def _iota16():
    return lax.broadcasted_iota(jnp.int32, (16,), 0)


@functools.partial(
    pl.kernel,
    mesh=_MESH,
    out_type=jax.ShapeDtypeStruct((2, KT), jnp.float32),
    scratch_types=[
        pltpu.VMEM((EPT,), jnp.int32),
        pltpu.VMEM((EPT,), jnp.int32),
        pltpu.VMEM((1, 128), jnp.int32),
        pltpu.VMEM((128,), jnp.float32),
        pltpu.VMEM((SL,), jnp.float32),
        pltpu.VMEM_SHARED((KT,), jnp.float32),
    ],
)
def _sc_cnt(et_hbm, dst_hbm, out_hbm, etb, dbb, kb, ones, zb, acc):
    c = lax.axis_index("c")
    s = lax.axis_index("s")
    wid = c * 16 + s

    def _zero(i, _):
        zb[pl.ds(i * 16, 16)] = jnp.zeros((16,), jnp.float32)
        return 0

    lax.fori_loop(0, SL // 16, _zero, 0)

    def _one(i, _):
        ones[pl.ds(i * 16, 16)] = jnp.full((16,), 1.0, jnp.float32)
        return 0

    lax.fori_loop(0, 8, _one, 0)

    pltpu.sync_copy(zb, acc.at[pl.ds(s * SL, SL)])
    plsc.subcore_barrier()

    base = wid * EPT
    pltpu.sync_copy(et_hbm.at[pl.ds(base, EPT)], etb)
    pltpu.sync_copy(dst_hbm.at[pl.ds(base, EPT)], dbb)

    def _chunk(j, _):
        for k in range(8):
            off = j * 128 + k * 16
            t16 = etb[pl.ds(off, 16)]
            d16 = dbb[pl.ds(off, 16)]
            kb[0, pl.ds(k * 16, 16)] = t16 * N + d16
        pltpu.sync_copy(ones, acc.at[kb.at[0]], add=True)
        return 0

    lax.fori_loop(0, FC, _chunk, 0)

    toff = FC * 128
    t16 = etb[pl.ds(toff, 16)]
    d16 = dbb[pl.ds(toff, 16)]
    kb[0, pl.ds(0, 16)] = t16 * N + d16
    for k in range(1, 8):
        kb[0, pl.ds(k * 16, 16)] = jnp.full((16,), R * N + k * 16, jnp.int32) + _iota16()
    pltpu.sync_copy(ones, acc.at[kb.at[0]], add=True)

    plsc.subcore_barrier()
    pltpu.sync_copy(acc.at[pl.ds(s * SL, SL)], out_hbm.at[c, pl.ds(s * SL, SL)])


@functools.partial(
    pl.kernel,
    mesh=_MESH,
    out_type=(
        jax.ShapeDtypeStruct((NTILES, NCH, 128), jnp.int32),
        jax.ShapeDtypeStruct((NTILES, NCH, 128), jnp.int32),
        jax.ShapeDtypeStruct((NTILES, NCH, 128), jnp.float32),
    ),
    scratch_types=[
        pltpu.VMEM((EPT,), jnp.int32),
        pltpu.VMEM((EPT,), jnp.int32),
        pltpu.VMEM((EPT,), jnp.int32),
        pltpu.VMEM((KT,), jnp.float32),
        pltpu.VMEM((SL,), jnp.float32),
        pltpu.VMEM((SL,), jnp.float32),
        pltpu.VMEM((128,), jnp.int32),
        pltpu.VMEM((128,), jnp.int32),
        pltpu.VMEM((128,), jnp.float32),
        pltpu.VMEM_SHARED((KT,), jnp.float32),
    ],
)
def _sc_prep(cnt_hbm, et_hbm, src_hbm, dst_hbm, gk_out, d_out, w_out,
             etb, sbb, dbb, invt, ca, cb, gkv, dbv, wv, shtab):
    c = lax.axis_index("c")
    s = lax.axis_index("s")
    wid = c * 16 + s

    pltpu.sync_copy(cnt_hbm.at[0, pl.ds(s * SL, SL)], ca)
    pltpu.sync_copy(cnt_hbm.at[1, pl.ds(s * SL, SL)], cb)

    def _inv(i, _):
        v = ca[pl.ds(i * 16, 16)] + cb[pl.ds(i * 16, 16)]
        ca[pl.ds(i * 16, 16)] = 1.0 / jnp.maximum(v, 1.0)
        return 0

    lax.fori_loop(0, SL // 16, _inv, 0)
    pltpu.sync_copy(ca, shtab.at[pl.ds(s * SL, SL)])
    plsc.subcore_barrier()
    pltpu.sync_copy(shtab, invt)

    base = wid * EPT
    pltpu.sync_copy(et_hbm.at[pl.ds(base, EPT)], etb)
    pltpu.sync_copy(src_hbm.at[pl.ds(base, EPT)], sbb)
    pltpu.sync_copy(dst_hbm.at[pl.ds(base, EPT)], dbb)

    def _emit16(off, k):
        t16 = etb[pl.ds(off, 16)]
        s16 = sbb[pl.ds(off, 16)]
        d16 = dbb[pl.ds(off, 16)]
        gkv[pl.ds(k * 16, 16)] = t16 * N + s16
        dbv[pl.ds(k * 16, 16)] = d16
        wv[pl.ds(k * 16, 16)] = plsc.load_gather(invt, [t16 * N + d16])

    def _chunk(j, _):
        for k in range(8):
            _emit16(j * 128 + k * 16, k)
        pltpu.sync_copy(gkv, gk_out.at[wid, j])
        pltpu.sync_copy(dbv, d_out.at[wid, j])
        pltpu.sync_copy(wv, w_out.at[wid, j])
        return 0

    lax.fori_loop(0, FC, _chunk, 0)

    _emit16(FC * 128, 0)
    for k in range(1, 8):
        spread = jnp.full((16,), wid * 112 + (k - 1) * 16, jnp.int32) + _iota16()
        gkv[pl.ds(k * 16, 16)] = spread
        dbv[pl.ds(k * 16, 16)] = spread
        wv[pl.ds(k * 16, 16)] = jnp.zeros((16,), jnp.float32)
    pltpu.sync_copy(gkv, gk_out.at[wid, FC])
    pltpu.sync_copy(dbv, d_out.at[wid, FC])
    pltpu.sync_copy(wv, w_out.at[wid, FC])


@functools.partial(
    pl.kernel,
    mesh=_MESH,
    out_type=jax.ShapeDtypeStruct((2, N, D), jnp.float32),
    scratch_types=[
        pltpu.VMEM((NCH, 128), jnp.int32),
        pltpu.VMEM((NCH, 128), jnp.int32),
        pltpu.VMEM((NCH, 128), jnp.float32),
        pltpu.VMEM((128, D), jnp.float32),
        pltpu.VMEM((125, D), jnp.float32),
        pltpu.VMEM_SHARED((N, D), jnp.float32),
        pltpu.SemaphoreType.DMA,
    ],
)
def _sc_msg(hall_hbm, gk_hbm, d_hbm, w_hbm, out_hbm,
            gkt, dstt, wt, rows, zb, acc, sem):
    c = lax.axis_index("c")
    s = lax.axis_index("s")
    wid = c * 16 + s

    def _zero(i, _):
        for k in range(8):
            zb[i, pl.ds(k * 16, 16)] = jnp.zeros((16,), jnp.float32)
        return 0

    lax.fori_loop(0, 125, _zero, 0)
    for m in range(5):
        pltpu.sync_copy(zb, acc.at[pl.ds(s * 625 + m * 125, 125)])

    pltpu.sync_copy(gk_hbm.at[wid], gkt)
    pltpu.sync_copy(d_hbm.at[wid], dstt)
    pltpu.sync_copy(w_hbm.at[wid], wt)
    plsc.subcore_barrier()

    def _chunk(j, _):
        pltpu.async_copy(hall_hbm.at[gkt.at[j]], rows, sem).wait()

        def _scale(i, _2):
            wspl = plsc.load_gather(
                wt, [jnp.zeros((16,), jnp.int32) + j, jnp.zeros((16,), jnp.int32) + i])
            for k in range(8):
                rows[i, pl.ds(k * 16, 16)] = rows[i, pl.ds(k * 16, 16)] * wspl
            return 0

        lax.fori_loop(0, 128, _scale, 0)
        pltpu.sync_copy(rows, acc.at[dstt.at[j]], add=True)
        return 0

    lax.fori_loop(0, NCH, _chunk, 0)

    plsc.subcore_barrier()
    for m in range(5):
        pltpu.sync_copy(acc.at[pl.ds(s * 625 + m * 125, 125)],
                        out_hbm.at[c, pl.ds(s * 625 + m * 125, 125)])


BN = 1000


def _build_w(comp_ref, bases_ref, r):
    w = comp_ref[r, 0] * bases_ref[0]
    for b in range(1, NB):
        w = w + comp_ref[r, b] * bases_ref[b]
    return w


def _tc_h1_body(x_ref, comp_ref, bases_ref, hall_ref):
    xv = x_ref[...]
    for r in range(R):
        hall_ref[r] = jnp.dot(xv, _build_w(comp_ref, bases_ref, r),
                              preferred_element_type=jnp.float32)


def _tc_h1(x, comp, bases):
    return pl.pallas_call(
        _tc_h1_body,
        grid=(N // BN,),
        in_specs=[
            pl.BlockSpec((BN, D), lambda i: (i, 0)),
            pl.BlockSpec((R, NB), lambda i: (0, 0)),
            pl.BlockSpec((NB, D, D), lambda i: (0, 0, 0)),
        ],
        out_specs=pl.BlockSpec((R, BN, D), lambda i: (0, i, 0)),
        out_shape=jax.ShapeDtypeStruct((R, N, D), jnp.float32),
    )(x, comp, bases)


def _tc_layer_body(x_ref, pa_ref, pb_ref, root_ref, bias_ref, comp_ref,
                   bases_ref, xn_ref, hall_ref):
    xn = jnp.dot(x_ref[...], root_ref[...], preferred_element_type=jnp.float32)
    xn = xn + bias_ref[...] + pa_ref[...] + pb_ref[...]
    xn = jnp.maximum(xn, 0.0)
    xn_ref[...] = xn
    for r in range(R):
        hall_ref[r] = jnp.dot(xn, _build_w(comp_ref, bases_ref, r),
                              preferred_element_type=jnp.float32)


def _tc_layer(x, pa, pb, root, bias, comp, bases):
    return pl.pallas_call(
        _tc_layer_body,
        grid=(N // BN,),
        in_specs=[
            pl.BlockSpec((BN, D), lambda i: (i, 0)),
            pl.BlockSpec((BN, D), lambda i: (i, 0)),
            pl.BlockSpec((BN, D), lambda i: (i, 0)),
            pl.BlockSpec((D, D), lambda i: (0, 0)),
            pl.BlockSpec((1, D), lambda i: (0, 0)),
            pl.BlockSpec((R, NB), lambda i: (0, 0)),
            pl.BlockSpec((NB, D, D), lambda i: (0, 0, 0)),
        ],
        out_specs=(
            pl.BlockSpec((BN, D), lambda i: (i, 0)),
            pl.BlockSpec((R, BN, D), lambda i: (0, i, 0)),
        ),
        out_shape=(
            jax.ShapeDtypeStruct((N, D), jnp.float32),
            jax.ShapeDtypeStruct((R, N, D), jnp.float32),
        ),
    )(x, pa, pb, root, bias, comp, bases)


def _tc_final_body(x_ref, pa_ref, pb_ref, root_ref, bias_ref, batch_ref,
                   rl_ref, rel_emb_ref, lw1_ref, lw2_ref, lb_ref, out_ref,
                   pooled, cnt):
    i = pl.program_id(0)

    @pl.when(i == 0)
    def _init():
        pooled[...] = jnp.zeros((G, D), jnp.float32)
        cnt[...] = jnp.zeros((G, D), jnp.float32)

    x4 = jnp.dot(x_ref[...], root_ref[...], preferred_element_type=jnp.float32)
    x4 = x4 + bias_ref[...] + pa_ref[...] + pb_ref[...]

    bb = batch_ref[0]
    gi = lax.broadcasted_iota(jnp.int32, (G, BN), 0)
    onehot = jnp.where(bb == gi, 1.0, 0.0).astype(jnp.float32)
    pooled[...] += jnp.dot(onehot, x4, preferred_element_type=jnp.float32)
    cnt[...] += jnp.broadcast_to(jnp.sum(onehot, axis=1, keepdims=True), (G, D))

    @pl.when(i == (N // BN) - 1)
    def _fin():
        pm = pooled[...] / jnp.maximum(cnt[...], 1.0)
        ri = lax.broadcasted_iota(jnp.int32, (G, R), 1)
        roh = jnp.where(rl_ref[...] == ri, 1.0, 0.0).astype(jnp.float32)
        rele = jnp.dot(roh, rel_emb_ref[...], preferred_element_type=jnp.float32)
        out = (jnp.dot(pm, lw1_ref[...], preferred_element_type=jnp.float32)
               + jnp.dot(rele, lw2_ref[...], preferred_element_type=jnp.float32)
               + lb_ref[...])
        out_ref[...] = out


def _tc_final(x, pa, pb, root, bias, batch3, rl2, rel_emb, lw1, lw2, lb2):
    return pl.pallas_call(
        _tc_final_body,
        grid=(N // BN,),
        in_specs=[
            pl.BlockSpec((BN, D), lambda i: (i, 0)),
            pl.BlockSpec((BN, D), lambda i: (i, 0)),
            pl.BlockSpec((BN, D), lambda i: (i, 0)),
            pl.BlockSpec((D, D), lambda i: (0, 0)),
            pl.BlockSpec((1, D), lambda i: (0, 0)),
            pl.BlockSpec((1, 1, BN), lambda i: (i, 0, 0)),
            pl.BlockSpec((G, 1), lambda i: (0, 0)),
            pl.BlockSpec((R, D), lambda i: (0, 0)),
            pl.BlockSpec((D, 2), lambda i: (0, 0)),
            pl.BlockSpec((D, 2), lambda i: (0, 0)),
            pl.BlockSpec((1, 2), lambda i: (0, 0)),
        ],
        out_specs=pl.BlockSpec((G, 2), lambda i: (0, 0)),
        out_shape=jax.ShapeDtypeStruct((G, 2), jnp.float32),
        scratch_shapes=[
            pltpu.VMEM((G, D), jnp.float32),
            pltpu.VMEM((G, D), jnp.float32),
        ],
    )(x, pa, pb, root, bias, batch3, rl2, rel_emb, lw1, lw2, lb2)


def kernel(x, edge_index, edge_type, batch, rel_labels, drop_prob,
           bases1, comp1, root1, bias1,
           bases2, comp2, root2, bias2,
           bases3, comp3, root3, bias3,
           rel_emb, lin_w, lin_b):
    src = edge_index[0]
    dst = edge_index[1]

    cnt2 = _sc_cnt(edge_type, dst)
    gkeyp, dstp, wp = _sc_prep(cnt2, edge_type, src, dst)

    hall = _tc_h1(x, comp1, bases1)
    p = _sc_msg(hall.reshape(R * N, D), gkeyp, dstp, wp)
    x2, hall = _tc_layer(x, p[0], p[1], root1, bias1.reshape(1, D),
                         comp2, bases2)
    p = _sc_msg(hall.reshape(R * N, D), gkeyp, dstp, wp)
    x3, hall = _tc_layer(x2, p[0], p[1], root2, bias2.reshape(1, D),
                         comp3, bases3)
    p = _sc_msg(hall.reshape(R * N, D), gkeyp, dstp, wp)

    out = _tc_final(x3, p[0], p[1], root3, bias3.reshape(1, D),
                    batch.reshape(N // BN, 1, BN),
                    rel_labels.reshape(G, 1), rel_emb,
                    lin_w[:D], lin_w[D:], lin_b.reshape(1, 2))
    return out


# speedup vs baseline: 23.6305x; 23.6305x over previous
import functools

import jax
import jax.numpy as jnp
from jax import lax
from jax.experimental import pallas as pl
from jax.experimental.pallas import tpu as pltpu
from jax.experimental.pallas import tpu_sc as plsc

N = 10000
E = 320000
R = 8
D = 128
G = 64
NB = 4
NTILES = 32
EPT = E // NTILES
NCH = 80
KT = R * N + 128
KTR = KT // 128
SL = KT // 16
NPAD = NCH * 128 - EPT

_MESH = plsc.VectorSubcoreMesh(core_axis_name="c", subcore_axis_name="s")


def _iota16():
    return lax.broadcasted_iota(jnp.int32, (16,), 0)


@functools.partial(
    pl.kernel,
    mesh=_MESH,
    compiler_params=pltpu.CompilerParams(
        use_tc_tiling_on_sc=False, needs_layout_passes=False),
    out_type=jax.ShapeDtypeStruct((2 * KT,), jnp.float32),
    scratch_types=[
        pltpu.VMEM((EPT,), jnp.int32),
        pltpu.VMEM((EPT,), jnp.int32),
        pltpu.VMEM((1, 128), jnp.int32),
        pltpu.VMEM((128,), jnp.float32),
        pltpu.VMEM((SL,), jnp.float32),
        pltpu.VMEM_SHARED((KT,), jnp.float32),
    ],
)
def _sc_cnt(et_hbm, dst_hbm, out_hbm, etb, dbb, kb, ones, zb, acc):
    c = lax.axis_index("c")
    s = lax.axis_index("s")
    wid = c * 16 + s

    def _zero(i, _):
        zb[pl.ds(i * 16, 16)] = jnp.zeros((16,), jnp.float32)
        return 0

    lax.fori_loop(0, SL // 16, _zero, 0)

    for k in range(8):
        ones[pl.ds(k * 16, 16)] = jnp.full((16,), 1.0, jnp.float32)

    pltpu.sync_copy(zb, acc.at[pl.ds(s * SL, SL)])
    plsc.subcore_barrier()

    base = wid * EPT
    pltpu.sync_copy(et_hbm.at[pl.ds(base, EPT)], etb)
    pltpu.sync_copy(dst_hbm.at[pl.ds(base, EPT)], dbb)

    def _chunk(j, _):
        for k in range(8):
            off = j * 128 + k * 16
            t16 = etb[pl.ds(off, 16)]
            d16 = dbb[pl.ds(off, 16)]
            kb[0, pl.ds(k * 16, 16)] = t16 * N + d16
        pltpu.sync_copy(ones, acc.at[kb.at[0]], add=True)
        return 0

    lax.fori_loop(0, EPT // 128, _chunk, 0)

    toff = (EPT // 128) * 128
    t16 = etb[pl.ds(toff, 16)]
    d16 = dbb[pl.ds(toff, 16)]
    kb[0, pl.ds(0, 16)] = t16 * N + d16
    for k in range(1, 8):
        kb[0, pl.ds(k * 16, 16)] = jnp.full((16,), R * N + k * 16, jnp.int32) + _iota16()
    pltpu.sync_copy(ones, acc.at[kb.at[0]], add=True)

    plsc.subcore_barrier()
    pltpu.sync_copy(acc.at[pl.ds(s * SL, SL)], zb)
    pltpu.sync_copy(zb, out_hbm.at[pl.ds(c * KT + s * SL, SL)])


def _tc_inv_body(c_ref, o_ref):
    o_ref[...] = 1.0 / jnp.maximum(c_ref[0] + c_ref[1], 1.0)


def _tc_inv(cnt2):
    return pl.pallas_call(
        _tc_inv_body,
        out_shape=jax.ShapeDtypeStruct((KTR, 128), jnp.float32),
    )(cnt2)


@functools.partial(
    pl.kernel,
    mesh=_MESH,
    compiler_params=pltpu.CompilerParams(
        use_tc_tiling_on_sc=False, needs_layout_passes=False),
    out_type=(
        jax.ShapeDtypeStruct((NTILES, NCH, 128), jnp.int32),
        jax.ShapeDtypeStruct((NTILES, NCH, 128), jnp.int32),
        jax.ShapeDtypeStruct((NTILES, NCH, 128), jnp.float32),
    ),
    scratch_types=[
        pltpu.VMEM((KT,), jnp.float32),
        pltpu.VMEM((2000,), jnp.int32),
        pltpu.VMEM((2000,), jnp.int32),
        pltpu.VMEM((2000,), jnp.int32),
        pltpu.VMEM((NCH, 128), jnp.int32),
        pltpu.VMEM((NCH, 128), jnp.int32),
        pltpu.VMEM((NCH, 128), jnp.float32),
    ],
)
def _sc_prep(inv_hbm, et_hbm, src_hbm, dst_hbm, gk_out, d_out, w_out,
             invt, etb, sbb, dbb, gkb, dstb, wvb):
    c = lax.axis_index("c")
    s = lax.axis_index("s")
    wid = c * 16 + s

    pltpu.sync_copy(inv_hbm, invt)

    for h in range(5):
        base = wid * EPT + h * 2000
        pltpu.sync_copy(et_hbm.at[pl.ds(base, 2000)], etb)
        pltpu.sync_copy(src_hbm.at[pl.ds(base, 2000)], sbb)
        pltpu.sync_copy(dst_hbm.at[pl.ds(base, 2000)], dbb)

        def _grp(t, _):
            gt = h * 125 + t
            row = gt // 8
            col = (gt % 8) * 16
            t16 = etb[pl.ds(t * 16, 16)]
            s16 = sbb[pl.ds(t * 16, 16)]
            d16 = dbb[pl.ds(t * 16, 16)]
            gkb[row, pl.ds(col, 16)] = t16 * N + s16
            dstb[row, pl.ds(col, 16)] = d16
            wvb[row, pl.ds(col, 16)] = plsc.load_gather(invt, [t16 * N + d16])
            return 0

        lax.fori_loop(0, 125, _grp, 0)

    pidx = 0
    for row, k0 in ((NCH - 2, 1), (NCH - 1, 0)):
        for k in range(k0, 8):
            spread = jnp.full((16,), wid * NPAD + pidx, jnp.int32) + _iota16()
            gkb[row, pl.ds(k * 16, 16)] = spread
            dstb[row, pl.ds(k * 16, 16)] = spread
            wvb[row, pl.ds(k * 16, 16)] = jnp.zeros((16,), jnp.float32)
            pidx += 16

    pltpu.sync_copy(gkb, gk_out.at[wid])
    pltpu.sync_copy(dstb, d_out.at[wid])
    pltpu.sync_copy(wvb, w_out.at[wid])


_ZR = 16


@functools.partial(
    pl.kernel,
    mesh=_MESH,
    compiler_params=pltpu.CompilerParams(
        use_tc_tiling_on_sc=False, needs_layout_passes=False),
    out_type=jax.ShapeDtypeStruct((2, N, D), jnp.float32),
    scratch_types=[
        pltpu.VMEM((NCH, 128), jnp.int32),
        pltpu.VMEM((NCH, 128), jnp.int32),
        pltpu.VMEM((NCH, 128), jnp.float32),
        pltpu.VMEM((128, D), jnp.float32),
        pltpu.VMEM((_ZR, D), jnp.float32),
        pltpu.VMEM_SHARED((N, D), jnp.float32),
        pltpu.SemaphoreType.DMA,
    ],
)
def _sc_msg(hall_hbm, gk_hbm, d_hbm, w_hbm, out_hbm,
            gkt, dstt, wt, rows, zb, acc, sem):
    c = lax.axis_index("c")
    s = lax.axis_index("s")
    wid = c * 16 + s

    def _zero(i, _):
        for k in range(8):
            zb[i, pl.ds(k * 16, 16)] = jnp.zeros((16,), jnp.float32)
        return 0

    lax.fori_loop(0, _ZR, _zero, 0)

    row0 = s * 624
    for m in range(39):
        pltpu.sync_copy(zb, acc.at[pl.ds(row0 + m * _ZR, _ZR)])

    @pl.when(s == 15)
    def _ztail():
        pltpu.sync_copy(zb.at[pl.ds(0, 16)], acc.at[pl.ds(9984, 16)])

    pltpu.sync_copy(gk_hbm.at[wid], gkt)
    pltpu.sync_copy(d_hbm.at[wid], dstt)
    pltpu.sync_copy(w_hbm.at[wid], wt)
    plsc.subcore_barrier()

    def _chunk(j, _):
        pltpu.async_copy(hall_hbm.at[gkt.at[j]], rows, sem).wait()

        def _scale(i, _2):
            wspl = plsc.load_gather(
                wt, [jnp.zeros((16,), jnp.int32) + j,
                     jnp.zeros((16,), jnp.int32) + i])
            for k in range(8):
                rows[i, pl.ds(k * 16, 16)] = rows[i, pl.ds(k * 16, 16)] * wspl
            return 0

        lax.fori_loop(0, 128, _scale, 0)
        pltpu.sync_copy(rows, acc.at[dstt.at[j]], add=True)
        return 0

    lax.fori_loop(0, NCH, _chunk, 0)

    plsc.subcore_barrier()
    for m in range(39):
        pltpu.sync_copy(acc.at[pl.ds(row0 + m * _ZR, _ZR)], zb)
        pltpu.sync_copy(zb, out_hbm.at[c, pl.ds(row0 + m * _ZR, _ZR)])

    @pl.when(s == 15)
    def _dtail():
        pltpu.sync_copy(acc.at[pl.ds(9984, 16)], zb.at[pl.ds(0, 16)])
        pltpu.sync_copy(zb.at[pl.ds(0, 16)], out_hbm.at[c, pl.ds(9984, 16)])


BN = 1000


def _build_w(comp_ref, bases_ref, r):
    w = comp_ref[r, 0] * bases_ref[0]
    for b in range(1, NB):
        w = w + comp_ref[r, b] * bases_ref[b]
    return w


def _tc_h1_body(x_ref, comp_ref, bases_ref, hall_ref):
    xv = x_ref[...]
    for r in range(R):
        hall_ref[r] = jnp.dot(xv, _build_w(comp_ref, bases_ref, r),
                              preferred_element_type=jnp.float32)


def _tc_h1(x, comp, bases):
    return pl.pallas_call(
        _tc_h1_body,
        grid=(N // BN,),
        in_specs=[
            pl.BlockSpec((BN, D), lambda i: (i, 0)),
            pl.BlockSpec((R, NB), lambda i: (0, 0)),
            pl.BlockSpec((NB, D, D), lambda i: (0, 0, 0)),
        ],
        out_specs=pl.BlockSpec((R, BN, D), lambda i: (0, i, 0)),
        out_shape=jax.ShapeDtypeStruct((R, N, D), jnp.float32),
    )(x, comp, bases)


def _tc_layer_body(x_ref, pa_ref, pb_ref, root_ref, bias_ref, comp_ref,
                   bases_ref, xn_ref, hall_ref):
    xn = jnp.dot(x_ref[...], root_ref[...], preferred_element_type=jnp.float32)
    xn = xn + bias_ref[...] + pa_ref[...] + pb_ref[...]
    xn = jnp.maximum(xn, 0.0)
    xn_ref[...] = xn
    for r in range(R):
        hall_ref[r] = jnp.dot(xn, _build_w(comp_ref, bases_ref, r),
                              preferred_element_type=jnp.float32)


def _tc_layer(x, pa, pb, root, bias, comp, bases):
    return pl.pallas_call(
        _tc_layer_body,
        grid=(N // BN,),
        in_specs=[
            pl.BlockSpec((BN, D), lambda i: (i, 0)),
            pl.BlockSpec((BN, D), lambda i: (i, 0)),
            pl.BlockSpec((BN, D), lambda i: (i, 0)),
            pl.BlockSpec((D, D), lambda i: (0, 0)),
            pl.BlockSpec((1, D), lambda i: (0, 0)),
            pl.BlockSpec((R, NB), lambda i: (0, 0)),
            pl.BlockSpec((NB, D, D), lambda i: (0, 0, 0)),
        ],
        out_specs=(
            pl.BlockSpec((BN, D), lambda i: (i, 0)),
            pl.BlockSpec((R, BN, D), lambda i: (0, i, 0)),
        ),
        out_shape=(
            jax.ShapeDtypeStruct((N, D), jnp.float32),
            jax.ShapeDtypeStruct((R, N, D), jnp.float32),
        ),
    )(x, pa, pb, root, bias, comp, bases)


def _tc_final_body(x_ref, pa_ref, pb_ref, root_ref, bias_ref, batch_ref,
                   rl_ref, rel_emb_ref, lw1_ref, lw2_ref, lb_ref, out_ref,
                   pooled, cnt):
    i = pl.program_id(0)

    @pl.when(i == 0)
    def _init():
        pooled[...] = jnp.zeros((G, D), jnp.float32)
        cnt[...] = jnp.zeros((G, D), jnp.float32)

    x4 = jnp.dot(x_ref[...], root_ref[...], preferred_element_type=jnp.float32)
    x4 = x4 + bias_ref[...] + pa_ref[...] + pb_ref[...]

    bb = batch_ref[0]
    gi = lax.broadcasted_iota(jnp.int32, (G, BN), 0)
    onehot = jnp.where(bb == gi, 1.0, 0.0).astype(jnp.float32)
    pooled[...] += jnp.dot(onehot, x4, preferred_element_type=jnp.float32)
    cnt[...] += jnp.broadcast_to(jnp.sum(onehot, axis=1, keepdims=True), (G, D))

    @pl.when(i == (N // BN) - 1)
    def _fin():
        pm = pooled[...] / jnp.maximum(cnt[...], 1.0)
        ri = lax.broadcasted_iota(jnp.int32, (G, R), 1)
        roh = jnp.where(rl_ref[...] == ri, 1.0, 0.0).astype(jnp.float32)
        rele = jnp.dot(roh, rel_emb_ref[...], preferred_element_type=jnp.float32)
        out = (jnp.dot(pm, lw1_ref[...], preferred_element_type=jnp.float32)
               + jnp.dot(rele, lw2_ref[...], preferred_element_type=jnp.float32)
               + lb_ref[...])
        out_ref[...] = out


def _tc_final(x, pa, pb, root, bias, batch3, rl2, rel_emb, lw1, lw2, lb2):
    return pl.pallas_call(
        _tc_final_body,
        grid=(N // BN,),
        in_specs=[
            pl.BlockSpec((BN, D), lambda i: (i, 0)),
            pl.BlockSpec((BN, D), lambda i: (i, 0)),
            pl.BlockSpec((BN, D), lambda i: (i, 0)),
            pl.BlockSpec((D, D), lambda i: (0, 0)),
            pl.BlockSpec((1, D), lambda i: (0, 0)),
            pl.BlockSpec((1, 1, BN), lambda i: (i, 0, 0)),
            pl.BlockSpec((G, 1), lambda i: (0, 0)),
            pl.BlockSpec((R, D), lambda i: (0, 0)),
            pl.BlockSpec((D, 2), lambda i: (0, 0)),
            pl.BlockSpec((D, 2), lambda i: (0, 0)),
            pl.BlockSpec((1, 2), lambda i: (0, 0)),
        ],
        out_specs=pl.BlockSpec((G, 2), lambda i: (0, 0)),
        out_shape=jax.ShapeDtypeStruct((G, 2), jnp.float32),
        scratch_shapes=[
            pltpu.VMEM((G, D), jnp.float32),
            pltpu.VMEM((G, D), jnp.float32),
        ],
    )(x, pa, pb, root, bias, batch3, rl2, rel_emb, lw1, lw2, lb2)


def kernel(x, edge_index, edge_type, batch, rel_labels, drop_prob,
           bases1, comp1, root1, bias1,
           bases2, comp2, root2, bias2,
           bases3, comp3, root3, bias3,
           rel_emb, lin_w, lin_b):
    src = edge_index[0]
    dst = edge_index[1]

    cnt2 = _sc_cnt(edge_type, dst)
    inv = _tc_inv(cnt2.reshape(2, KTR, 128))
    gkeyp, dstp, wp = _sc_prep(inv.reshape(KT), edge_type, src, dst)

    hall = _tc_h1(x, comp1, bases1)
    p = _sc_msg(hall.reshape(R * N, D), gkeyp, dstp, wp)
    x2, hall = _tc_layer(x, p[0], p[1], root1, bias1.reshape(1, D),
                         comp2, bases2)
    p = _sc_msg(hall.reshape(R * N, D), gkeyp, dstp, wp)
    x3, hall = _tc_layer(x2, p[0], p[1], root2, bias2.reshape(1, D),
                         comp3, bases3)
    p = _sc_msg(hall.reshape(R * N, D), gkeyp, dstp, wp)

    out = _tc_final(x3, p[0], p[1], root3, bias3.reshape(1, D),
                    batch.reshape(N // BN, 1, BN),
                    rel_labels.reshape(G, 1), rel_emb,
                    lin_w[:D], lin_w[D:], lin_b.reshape(1, 2))
    return out
